# Initial kernel scaffold; baseline (speedup 1.0000x reference)
import jax
import jax.numpy as jnp
from jax.experimental import pallas as pl


def kernel(pred, target):
    raise NotImplementedError("write your pallas kernel here")


# speedup vs baseline: 7.1572x; 7.1572x over previous
import functools

import jax
import jax.numpy as jnp
from jax import lax
from jax.experimental import pallas as pl
from jax.experimental.pallas import tpu as pltpu
from jax.experimental.pallas import tpu_sc as plsc

_K = 0.15


def _nll_body(pred_ref, tgt_ref, out_ref):
    x = pred_ref[0]
    t = tgt_ref[0]
    m = jnp.max(x, axis=0)
    s = jnp.sum(jnp.exp(x - m[None]), axis=0)
    lse = m + jnp.log(s)
    cls = lax.broadcasted_iota(jnp.int32, x.shape, 0)
    xt = jnp.sum(jnp.where(cls == t[None], x, 0.0), axis=0)
    out_ref[0] = lse - xt


def _nll(pred, target):
    B, C, H, W = pred.shape
    R = 32
    return pl.pallas_call(
        _nll_body,
        grid=(B, H // R),
        in_specs=[
            pl.BlockSpec((1, C, R, W), lambda b, i: (b, 0, i, 0)),
            pl.BlockSpec((1, R, W), lambda b, i: (b, i, 0)),
        ],
        out_specs=pl.BlockSpec((1, R, W), lambda b, i: (b, i, 0)),
        out_shape=jax.ShapeDtypeStruct((B, H, W), jnp.float32),
    )(pred, target)


_NSUB = 16
_CHUNK = 16384
_NBINS = 2048


@functools.lru_cache(maxsize=None)
def _sc_select(N, num):
    per = N // _NSUB
    nch = per // _CHUNK
    assert per % _CHUNK == 0
    mesh = plsc.VectorSubcoreMesh(
        core_axis_name="c", subcore_axis_name="s", num_cores=1
    )

    def body(loss_hbm, out_hbm, data_v, hist_v, merged_v, acc_v, accall_v,
             outbuf_v, sh_hist, sh_acc):
        wid = lax.axis_index("s")
        base = wid * per
        iota = lax.iota(jnp.int32, 16)
        ones = jnp.ones((16,), jnp.int32)
        zeros16 = jnp.zeros((16,), jnp.int32)

        def lane_merge():
            def mbody(cc, _):
                acc = hist_v[0, pl.ds(cc * 16, 16)]
                for l in range(1, 16):
                    acc = acc + hist_v[l, pl.ds(cc * 16, 16)]
                merged_v[pl.ds(cc * 16, 16)] = acc
                return 0

            lax.fori_loop(0, _NBINS // 16, mbody, 0)

        def hist_pass(shift, nbits, sel_shift, sel_val, rprime):
            nb = 1 << nbits
            binmask = nb - 1

            def zbody(cc, _):
                for l in range(16):
                    hist_v[l, pl.ds(cc * 16, 16)] = zeros16
                return 0

            lax.fori_loop(0, _NBINS // 16, zbody, 0)

            def chunk_body(ch, _):
                pltpu.sync_copy(
                    loss_hbm.at[pl.ds(base + ch * _CHUNK, _CHUNK)], data_v
                )

                def vbody(j, _):
                    v = data_v[pl.ds(j * 16, 16)]
                    b = plsc.bitcast(v, jnp.int32)
                    binv = lax.shift_right_logical(b, shift) & binmask
                    if sel_val is None:
                        plsc.addupdate_scatter(hist_v, [iota, binv], ones)
                    else:
                        keym = (
                            lax.shift_right_logical(b, sel_shift) == sel_val
                        )
                        plsc.addupdate_scatter(
                            hist_v, [iota, binv], ones, mask=keym
                        )
                    return 0

                lax.fori_loop(0, _CHUNK // 16, vbody, 0)
                return 0

            lax.fori_loop(0, nch, chunk_body, 0)

            lane_merge()

            pltpu.sync_copy(merged_v, sh_hist.at[wid])
            plsc.subcore_barrier()
            pltpu.sync_copy(sh_hist, hist_v)
            plsc.subcore_barrier()
            lane_merge()

            def scan_body(jj, carry):
                acc, done, bin_, below, binval = carry
                vec = merged_v[pl.ds(jj * 16, 16)]
                svec = jnp.sum(vec)
                cum = plsc.cumsum(vec)
                a = acc + cum
                crossm = a >= rprime
                any_ = (acc + svec) >= rprime
                i0 = plsc.all_reduce_ffs(crossm)
                sel = iota == i0
                v_i0 = jnp.sum(jnp.where(sel, vec, 0))
                a_i0 = jnp.sum(jnp.where(sel, a, 0))
                lane = jnp.sum(jnp.where(sel, iota, 0))
                upd = jnp.logical_and(any_, done == 0)
                bin_ = jnp.where(upd, jj * 16 + lane, bin_)
                below = jnp.where(upd, a_i0 - v_i0, below)
                binval = jnp.where(upd, v_i0, binval)
                acc = acc + svec
                done = jnp.where(any_, jnp.int32(1), done)
                return acc, done, bin_, below, binval

            z = jnp.int32(0)
            _, _, bin_, below, binval = lax.fori_loop(
                0, nb // 16, scan_body, (z, z, z, z, z)
            )
            return bin_, below, binval

        r = jnp.int32(num)
        cnt = jnp.int32(N)

        b1, below, binval = hist_pass(20, 11, None, None, cnt - r + 1)
        r = r - (cnt - below - binval)
        cnt = binval

        b2, below, binval = hist_pass(10, 10, 20, b1, cnt - r + 1)
        r = r - (cnt - below - binval)
        cnt = binval

        b3, _, _ = hist_pass(0, 10, 10, (b1 << 10) | b2, cnt - r + 1)

        tk_bits = (b1 << 20) | (b2 << 10) | b3
        tkvec = plsc.bitcast(jnp.full((16,), tk_bits, jnp.int32), jnp.float32)

        def sum_chunk(ch, carry):
            s, c = carry
            pltpu.sync_copy(
                loss_hbm.at[pl.ds(base + ch * _CHUNK, _CHUNK)], data_v
            )

            def vbody(j, carry2):
                s2, c2 = carry2
                v = data_v[pl.ds(j * 16, 16)]
                m = v >= tkvec
                s2 = s2 + jnp.where(m, v, jnp.float32(0))
                c2 = c2 + jnp.where(m, jnp.float32(1), jnp.float32(0))
                return s2, c2

            return lax.fori_loop(0, _CHUNK // 16, vbody, (s, c))

        zf = jnp.zeros((16,), jnp.float32)
        s, c = lax.fori_loop(0, nch, sum_chunk, (zf, zf))
        acc_v[0, :] = s
        acc_v[1, :] = c
        pltpu.sync_copy(acc_v, sh_acc.at[wid])
        plsc.subcore_barrier()

        @pl.when(wid == 0)
        def _():
            pltpu.sync_copy(sh_acc, accall_v)
            st = jnp.zeros((16,), jnp.float32)
            ct = jnp.zeros((16,), jnp.float32)
            for w in range(_NSUB):
                st = st + accall_v[w, 0, :]
                ct = ct + accall_v[w, 1, :]
            num_vec = jnp.full((16,), jnp.sum(st), jnp.float32)
            den_vec = jnp.full((16,), jnp.sum(ct), jnp.float32)
            outbuf_v[...] = num_vec / den_vec
            pltpu.sync_copy(outbuf_v, out_hbm)

    return pl.kernel(
        body,
        out_type=jax.ShapeDtypeStruct((16,), jnp.float32),
        mesh=mesh,
        compiler_params=pltpu.CompilerParams(needs_layout_passes=False),
        scratch_types=[
            pltpu.VMEM((_CHUNK,), jnp.float32),
            pltpu.VMEM((16, _NBINS), jnp.int32),
            pltpu.VMEM((_NBINS,), jnp.int32),
            pltpu.VMEM((2, 16), jnp.float32),
            pltpu.VMEM((_NSUB, 2, 16), jnp.float32),
            pltpu.VMEM((16,), jnp.float32),
            pltpu.VMEM_SHARED((_NSUB, _NBINS), jnp.int32),
            pltpu.VMEM_SHARED((_NSUB, 2, 16), jnp.float32),
        ],
    )


def kernel(pred, target):
    B, C, H, W = pred.shape
    N = B * H * W
    num = int(_K * B * H * W)
    loss = _nll(pred, target)
    out = _sc_select(N, num)(loss.reshape(N))
    return out[0]


# speedup vs baseline: 7.4939x; 1.0470x over previous
import functools

import jax
import jax.numpy as jnp
from jax import lax
from jax.experimental import pallas as pl
from jax.experimental.pallas import tpu as pltpu
from jax.experimental.pallas import tpu_sc as plsc

_K = 0.15


def _nll_body(pred_ref, tgt_ref, out_ref):
    x = pred_ref[0]
    t = tgt_ref[0]
    m = jnp.max(x, axis=0)
    s = jnp.sum(jnp.exp(x - m[None]), axis=0)
    lse = m + jnp.log(s)
    cls = lax.broadcasted_iota(jnp.int32, x.shape, 0)
    xt = jnp.sum(jnp.where(cls == t[None], x, 0.0), axis=0)
    out_ref[0] = lse - xt


def _nll(pred, target):
    B, C, H, W = pred.shape
    R = 32
    return pl.pallas_call(
        _nll_body,
        grid=(B, H // R),
        in_specs=[
            pl.BlockSpec((1, C, R, W), lambda b, i: (b, 0, i, 0)),
            pl.BlockSpec((1, R, W), lambda b, i: (b, i, 0)),
        ],
        out_specs=pl.BlockSpec((1, R, W), lambda b, i: (b, i, 0)),
        out_shape=jax.ShapeDtypeStruct((B, H, W), jnp.float32),
    )(pred, target)


_NSUB = 16
_CHUNK = 16384
_NBINS = 2048
_U = 8


@functools.lru_cache(maxsize=None)
def _sc_select(N, num):
    per = N // _NSUB
    nch = per // _CHUNK
    assert per % _CHUNK == 0
    mesh = plsc.VectorSubcoreMesh(
        core_axis_name="c", subcore_axis_name="s", num_cores=1
    )

    def body(loss_hbm, out_hbm, data_v, hist_v, vhist_v, merged_v, mergedv_v,
             outbuf_v, sh_hist, sh_vhist):
        wid = lax.axis_index("s")
        base = wid * per
        iota = lax.iota(jnp.int32, 16)
        ones = jnp.ones((16,), jnp.int32)
        zeros16 = jnp.zeros((16,), jnp.int32)
        zf16 = jnp.zeros((16,), jnp.float32)

        def zero_hist(with_values):
            def zbody(cc, _):
                for l in range(16):
                    hist_v[l, pl.ds(cc * 16, 16)] = zeros16
                    if with_values:
                        vhist_v[l, pl.ds(cc * 16, 16)] = zf16
                return 0

            lax.fori_loop(0, _NBINS // 16, zbody, 0)

        def lane_merge(src, dst):
            def mbody(cc, _):
                acc = src[0, pl.ds(cc * 16, 16)]
                for l in range(1, 16):
                    acc = acc + src[l, pl.ds(cc * 16, 16)]
                dst[pl.ds(cc * 16, 16)] = acc
                return 0

            lax.fori_loop(0, _NBINS // 16, mbody, 0)

        def merge_tiles(with_values):
            lane_merge(hist_v, merged_v)
            if with_values:
                lane_merge(vhist_v, mergedv_v)
            pltpu.sync_copy(merged_v, sh_hist.at[wid])
            if with_values:
                pltpu.sync_copy(mergedv_v, sh_vhist.at[wid])
            plsc.subcore_barrier()
            pltpu.sync_copy(sh_hist, hist_v)
            if with_values:
                pltpu.sync_copy(sh_vhist, vhist_v)
            plsc.subcore_barrier()
            lane_merge(hist_v, merged_v)
            if with_values:
                lane_merge(vhist_v, mergedv_v)

        def scan(nb, rprime):
            def scan_body(jj, carry):
                acc, done, bin_, below, binval = carry
                vec = merged_v[pl.ds(jj * 16, 16)]
                svec = jnp.sum(vec)
                cum = plsc.cumsum(vec)
                a = acc + cum
                crossm = a >= rprime
                any_ = (acc + svec) >= rprime
                i0 = plsc.all_reduce_ffs(crossm)
                sel = iota == i0
                v_i0 = jnp.sum(jnp.where(sel, vec, 0))
                a_i0 = jnp.sum(jnp.where(sel, a, 0))
                lane = jnp.sum(jnp.where(sel, iota, 0))
                upd = jnp.logical_and(any_, done == 0)
                bin_ = jnp.where(upd, jj * 16 + lane, bin_)
                below = jnp.where(upd, a_i0 - v_i0, below)
                binval = jnp.where(upd, v_i0, binval)
                acc = acc + svec
                done = jnp.where(any_, jnp.int32(1), done)
                return acc, done, bin_, below, binval

            z = jnp.int32(0)
            _, _, bin_, below, binval = lax.fori_loop(
                0, nb // 16, scan_body, (z, z, z, z, z)
            )
            return bin_, below, binval

        def stream_loop(process):
            def chunk_body(ch, _):
                pltpu.sync_copy(
                    loss_hbm.at[pl.ds(base + ch * _CHUNK, _CHUNK)], data_v
                )

                def vbody(j, _):
                    for u in range(_U):
                        v = data_v[pl.ds((j * _U + u) * 16, 16)]
                        process(v)
                    return 0

                lax.fori_loop(0, _CHUNK // 16 // _U, vbody, 0)
                return 0

            lax.fori_loop(0, nch, chunk_body, 0)

        r = jnp.int32(num)
        cnt = jnp.int32(N)

        zero_hist(False)

        def p1(v):
            b = plsc.bitcast(v, jnp.int32)
            binv = lax.shift_right_logical(b, 20)
            plsc.addupdate_scatter(hist_v, [iota, binv], ones)

        stream_loop(p1)
        merge_tiles(False)
        b1, below, binval = scan(2048, cnt - r + 1)
        r = r - (cnt - below - binval)
        cnt = binval

        zero_hist(False)

        def p2(v):
            b = plsc.bitcast(v, jnp.int32)
            keym = lax.shift_right_logical(b, 20) == b1
            binv = lax.shift_right_logical(b, 10) & 1023
            plsc.addupdate_scatter(hist_v, [iota, binv], ones, mask=keym)

        stream_loop(p2)
        merge_tiles(False)
        b2, below, binval = scan(1024, cnt - r + 1)
        r = r - (cnt - below - binval)
        cnt = binval

        sel21 = (b1 << 10) | b2
        zero_hist(True)

        def p3(v):
            b = plsc.bitcast(v, jnp.int32)
            top21 = lax.shift_right_logical(b, 10)
            m = top21 >= sel21
            binv = jnp.where(top21 == sel21, b & 1023, jnp.int32(1024))
            plsc.addupdate_scatter(hist_v, [iota, binv], ones, mask=m)
            plsc.addupdate_scatter(vhist_v, [iota, binv], v, mask=m)

        stream_loop(p3)
        merge_tiles(True)
        b3, _, _ = scan(1024, cnt - r + 1)

        def suf_body(jj, carry):
            tc, pc, ts, ps = carry
            vec = merged_v[pl.ds(jj * 16, 16)]
            vvec = mergedv_v[pl.ds(jj * 16, 16)]
            inpref = (jj * 16 + iota) < b3
            tc = tc + vec
            pc = pc + jnp.where(inpref, vec, 0)
            ts = ts + vvec
            ps = ps + jnp.where(inpref, vvec, jnp.float32(0))
            return tc, pc, ts, ps

        tc, pc, ts, ps = lax.fori_loop(
            0, 1040 // 16, suf_body, (zeros16, zeros16, zf16, zf16)
        )
        c_tot = (jnp.sum(tc) - jnp.sum(pc)).astype(jnp.float32)
        s_tot = jnp.sum(ts) - jnp.sum(ps)

        @pl.when(wid == 0)
        def _():
            num_vec = jnp.full((16,), s_tot, jnp.float32)
            den_vec = jnp.full((16,), c_tot, jnp.float32)
            outbuf_v[...] = num_vec / den_vec
            pltpu.sync_copy(outbuf_v, out_hbm)

    return pl.kernel(
        body,
        out_type=jax.ShapeDtypeStruct((16,), jnp.float32),
        mesh=mesh,
        compiler_params=pltpu.CompilerParams(needs_layout_passes=False),
        scratch_types=[
            pltpu.VMEM((_CHUNK,), jnp.float32),
            pltpu.VMEM((16, _NBINS), jnp.int32),
            pltpu.VMEM((16, _NBINS), jnp.float32),
            pltpu.VMEM((_NBINS,), jnp.int32),
            pltpu.VMEM((_NBINS,), jnp.float32),
            pltpu.VMEM((16,), jnp.float32),
            pltpu.VMEM_SHARED((_NSUB, _NBINS), jnp.int32),
            pltpu.VMEM_SHARED((_NSUB, _NBINS), jnp.float32),
        ],
    )


def kernel(pred, target):
    B, C, H, W = pred.shape
    N = B * H * W
    num = int(_K * B * H * W)
    loss = _nll(pred, target)
    out = _sc_select(N, num)(loss.reshape(N))
    return out[0]


# speedup vs baseline: 11.5424x; 1.5402x over previous
import functools

import jax
import jax.numpy as jnp
from jax import lax
from jax.experimental import pallas as pl
from jax.experimental.pallas import tpu as pltpu
from jax.experimental.pallas import tpu_sc as plsc

_K = 0.15


def _nll_body(pred_ref, tgt_ref, out_ref):
    x = pred_ref[0]
    t = tgt_ref[0]
    m = jnp.max(x, axis=0)
    s = jnp.sum(jnp.exp(x - m[None]), axis=0)
    lse = m + jnp.log(s)
    cls = lax.broadcasted_iota(jnp.int32, x.shape, 0)
    xt = jnp.sum(jnp.where(cls == t[None], x, 0.0), axis=0)
    out_ref[0] = lse - xt


def _nll(pred, target):
    B, C, H, W = pred.shape
    R = 32
    return pl.pallas_call(
        _nll_body,
        grid=(B, H // R),
        in_specs=[
            pl.BlockSpec((1, C, R, W), lambda b, i: (b, 0, i, 0)),
            pl.BlockSpec((1, R, W), lambda b, i: (b, i, 0)),
        ],
        out_specs=pl.BlockSpec((1, R, W), lambda b, i: (b, i, 0)),
        out_shape=jax.ShapeDtypeStruct((B, H, W), jnp.float32),
    )(pred, target)


_NSUB = 16
_CHUNK = 16384
_NBINS = 2048
_U = 8


@functools.lru_cache(maxsize=None)
def _sc_select(N, num):
    per = N // _NSUB
    nch = per // _CHUNK
    assert per % _CHUNK == 0
    mesh = plsc.VectorSubcoreMesh(
        core_axis_name="c", subcore_axis_name="s", num_cores=1
    )

    def body(loss_hbm, out_hbm, data_v, hist_v, vhist_v, merged_v, mergedv_v,
             outbuf_v, sh_hist, sh_vhist):
        wid = lax.axis_index("s")
        base = wid * per
        iota = lax.iota(jnp.int32, 16)
        ones = jnp.ones((16,), jnp.int32)
        zeros16 = jnp.zeros((16,), jnp.int32)
        zf16 = jnp.zeros((16,), jnp.float32)

        def zero_hist(with_values):
            def zbody(cc, _):
                for l in range(16):
                    hist_v[l, pl.ds(cc * 16, 16)] = zeros16
                    if with_values:
                        vhist_v[l, pl.ds(cc * 16, 16)] = zf16
                return 0

            lax.fori_loop(0, _NBINS // 16, zbody, 0)

        def lane_merge(src, dst):
            def mbody(cc, _):
                acc = src[0, pl.ds(cc * 16, 16)]
                for l in range(1, 16):
                    acc = acc + src[l, pl.ds(cc * 16, 16)]
                dst[pl.ds(cc * 16, 16)] = acc
                return 0

            lax.fori_loop(0, _NBINS // 16, mbody, 0)

        def merge_tiles(with_values):
            lane_merge(hist_v, merged_v)
            if with_values:
                lane_merge(vhist_v, mergedv_v)
            pltpu.sync_copy(merged_v, sh_hist.at[wid])
            if with_values:
                pltpu.sync_copy(mergedv_v, sh_vhist.at[wid])
            plsc.subcore_barrier()
            pltpu.sync_copy(sh_hist, hist_v)
            if with_values:
                pltpu.sync_copy(sh_vhist, vhist_v)
            plsc.subcore_barrier()
            lane_merge(hist_v, merged_v)
            if with_values:
                lane_merge(vhist_v, mergedv_v)

        def scan(nb, rprime):
            def scan_body(jj, carry):
                acc, done, bin_, below, binval = carry
                vec = merged_v[pl.ds(jj * 16, 16)]
                svec = jnp.sum(vec)
                cum = plsc.cumsum(vec)
                a = acc + cum
                crossm = a >= rprime
                any_ = (acc + svec) >= rprime
                i0 = plsc.all_reduce_ffs(crossm)
                sel = iota == i0
                v_i0 = jnp.sum(jnp.where(sel, vec, 0))
                a_i0 = jnp.sum(jnp.where(sel, a, 0))
                lane = jnp.sum(jnp.where(sel, iota, 0))
                upd = jnp.logical_and(any_, done == 0)
                bin_ = jnp.where(upd, jj * 16 + lane, bin_)
                below = jnp.where(upd, a_i0 - v_i0, below)
                binval = jnp.where(upd, v_i0, binval)
                acc = acc + svec
                done = jnp.where(any_, jnp.int32(1), done)
                return acc, done, bin_, below, binval

            z = jnp.int32(0)
            _, _, bin_, below, binval = lax.fori_loop(
                0, nb // 16, scan_body, (z, z, z, z, z)
            )
            return bin_, below, binval

        def stream_loop(process_batch):
            def chunk_body(ch, _):
                pltpu.sync_copy(
                    loss_hbm.at[pl.ds(base + ch * _CHUNK, _CHUNK)], data_v
                )

                def vbody(j, _):
                    vals = [
                        data_v[pl.ds((j * _U + u) * 16, 16)]
                        for u in range(_U)
                    ]
                    process_batch(vals)
                    return 0

                lax.fori_loop(0, _CHUNK // 16 // _U, vbody, 0)
                return 0

            lax.fori_loop(0, nch, chunk_body, 0)

        r = jnp.int32(num)
        cnt = jnp.int32(N)

        zero_hist(False)

        def p1(vals):
            bins = []
            for v in vals:
                b = plsc.bitcast(v, jnp.int32)
                bins.append(lax.shift_right_logical(b, 20))
            for binv in bins:
                plsc.addupdate_scatter(hist_v, [iota, binv], ones)

        stream_loop(p1)
        merge_tiles(False)
        b1, below, binval = scan(2048, cnt - r + 1)
        r = r - (cnt - below - binval)
        cnt = binval

        zero_hist(False)

        def p2(vals):
            bins = []
            for v in vals:
                b = plsc.bitcast(v, jnp.int32)
                keym = lax.shift_right_logical(b, 20) == b1
                binv = lax.shift_right_logical(b, 10) & 1023
                bins.append((binv, keym))
            for binv, keym in bins:
                plsc.addupdate_scatter(hist_v, [iota, binv], ones, mask=keym)

        stream_loop(p2)
        merge_tiles(False)
        b2, below, binval = scan(1024, cnt - r + 1)
        r = r - (cnt - below - binval)
        cnt = binval

        sel21 = (b1 << 10) | b2
        zero_hist(True)

        def p3(vals):
            bins = []
            for v in vals:
                b = plsc.bitcast(v, jnp.int32)
                top21 = lax.shift_right_logical(b, 10)
                m = top21 >= sel21
                binv = jnp.where(top21 == sel21, b & 1023, jnp.int32(1024))
                bins.append((binv, m, v))
            for binv, m, v in bins:
                plsc.addupdate_scatter(hist_v, [iota, binv], ones, mask=m)
            for binv, m, v in bins:
                plsc.addupdate_scatter(vhist_v, [iota, binv], v, mask=m)

        stream_loop(p3)
        merge_tiles(True)
        b3, _, _ = scan(1024, cnt - r + 1)

        def suf_body(jj, carry):
            tc, pc, ts, ps = carry
            vec = merged_v[pl.ds(jj * 16, 16)]
            vvec = mergedv_v[pl.ds(jj * 16, 16)]
            inpref = (jj * 16 + iota) < b3
            tc = tc + vec
            pc = pc + jnp.where(inpref, vec, 0)
            ts = ts + vvec
            ps = ps + jnp.where(inpref, vvec, jnp.float32(0))
            return tc, pc, ts, ps

        tc, pc, ts, ps = lax.fori_loop(
            0, 1040 // 16, suf_body, (zeros16, zeros16, zf16, zf16)
        )
        c_tot = (jnp.sum(tc) - jnp.sum(pc)).astype(jnp.float32)
        s_tot = jnp.sum(ts) - jnp.sum(ps)

        @pl.when(wid == 0)
        def _():
            num_vec = jnp.full((16,), s_tot, jnp.float32)
            den_vec = jnp.full((16,), c_tot, jnp.float32)
            outbuf_v[...] = num_vec / den_vec
            pltpu.sync_copy(outbuf_v, out_hbm)

    return pl.kernel(
        body,
        out_type=jax.ShapeDtypeStruct((16,), jnp.float32),
        mesh=mesh,
        compiler_params=pltpu.CompilerParams(needs_layout_passes=False),
        scratch_types=[
            pltpu.VMEM((_CHUNK,), jnp.float32),
            pltpu.VMEM((16, _NBINS), jnp.int32),
            pltpu.VMEM((16, _NBINS), jnp.float32),
            pltpu.VMEM((_NBINS,), jnp.int32),
            pltpu.VMEM((_NBINS,), jnp.float32),
            pltpu.VMEM((16,), jnp.float32),
            pltpu.VMEM_SHARED((_NSUB, _NBINS), jnp.int32),
            pltpu.VMEM_SHARED((_NSUB, _NBINS), jnp.float32),
        ],
    )


def kernel(pred, target):
    B, C, H, W = pred.shape
    N = B * H * W
    num = int(_K * B * H * W)
    loss = _nll(pred, target)
    out = _sc_select(N, num)(loss.reshape(N))
    return out[0]


# speedup vs baseline: 12.5068x; 1.0836x over previous
import functools

import jax
import jax.numpy as jnp
from jax import lax
from jax.experimental import pallas as pl
from jax.experimental.pallas import tpu as pltpu
from jax.experimental.pallas import tpu_sc as plsc

_K = 0.15


def _nll_body(pred_ref, tgt_ref, out_ref):
    x = pred_ref[0]
    t = tgt_ref[0]
    m = jnp.max(x, axis=0)
    s = jnp.sum(jnp.exp(x - m[None]), axis=0)
    lse = m + jnp.log(s)
    cls = lax.broadcasted_iota(jnp.int32, x.shape, 0)
    xt = jnp.sum(jnp.where(cls == t[None], x, 0.0), axis=0)
    out_ref[0] = lse - xt


def _nll(pred, target):
    B, C, H, W = pred.shape
    R = 32
    return pl.pallas_call(
        _nll_body,
        grid=(B, H // R),
        in_specs=[
            pl.BlockSpec((1, C, R, W), lambda b, i: (b, 0, i, 0)),
            pl.BlockSpec((1, R, W), lambda b, i: (b, i, 0)),
        ],
        out_specs=pl.BlockSpec((1, R, W), lambda b, i: (b, i, 0)),
        out_shape=jax.ShapeDtypeStruct((B, H, W), jnp.float32),
    )(pred, target)


_NSUB = 16
_CHUNK = 16384
_NBINS = 2048
_U = 8


@functools.lru_cache(maxsize=None)
def _sc_select(N, num):
    per = N // _NSUB
    nch = per // _CHUNK
    assert per % _CHUNK == 0
    mesh = plsc.VectorSubcoreMesh(
        core_axis_name="c", subcore_axis_name="s", num_cores=1
    )

    def body(loss_hbm, out_hbm, data_v, hist_v, vhist_v, merged_v, mergedv_v,
             outbuf_v, sh_hist, sh_vhist, sem0, sem1):
        wid = lax.axis_index("s")
        base = wid * per
        iota = lax.iota(jnp.int32, 16)
        ones = jnp.ones((16,), jnp.int32)
        zeros16 = jnp.zeros((16,), jnp.int32)
        zf16 = jnp.zeros((16,), jnp.float32)

        def zero_hist(width, with_values):
            def zbody(cc, _):
                for l in range(16):
                    hist_v[l, pl.ds(cc * 16, 16)] = zeros16
                    if with_values:
                        vhist_v[l, pl.ds(cc * 16, 16)] = zf16
                return 0

            lax.fori_loop(0, width // 16, zbody, 0)

        def lane_merge(src, dst, width):
            def mbody(cc, _):
                acc = src[0, pl.ds(cc * 16, 16)]
                for l in range(1, 16):
                    acc = acc + src[l, pl.ds(cc * 16, 16)]
                dst[pl.ds(cc * 16, 16)] = acc
                return 0

            lax.fori_loop(0, width // 16, mbody, 0)

        def merge_tiles(width, with_values):
            lane_merge(hist_v, merged_v, width)
            if with_values:
                lane_merge(vhist_v, mergedv_v, width)
            pltpu.sync_copy(merged_v, sh_hist.at[wid])
            if with_values:
                pltpu.sync_copy(mergedv_v, sh_vhist.at[wid])
            plsc.subcore_barrier()
            pltpu.sync_copy(sh_hist, hist_v)
            if with_values:
                pltpu.sync_copy(sh_vhist, vhist_v)
            plsc.subcore_barrier()
            lane_merge(hist_v, merged_v, width)
            if with_values:
                lane_merge(vhist_v, mergedv_v, width)

        def scan(nb, rprime):
            def scan_body(jj, carry):
                acc, done, bin_, below, binval = carry
                vec = merged_v[pl.ds(jj * 16, 16)]
                svec = jnp.sum(vec)
                cum = plsc.cumsum(vec)
                a = acc + cum
                crossm = a >= rprime
                any_ = (acc + svec) >= rprime
                i0 = plsc.all_reduce_ffs(crossm)
                sel = iota == i0
                v_i0 = jnp.sum(jnp.where(sel, vec, 0))
                a_i0 = jnp.sum(jnp.where(sel, a, 0))
                lane = jnp.sum(jnp.where(sel, iota, 0))
                upd = jnp.logical_and(any_, done == 0)
                bin_ = jnp.where(upd, jj * 16 + lane, bin_)
                below = jnp.where(upd, a_i0 - v_i0, below)
                binval = jnp.where(upd, v_i0, binval)
                acc = acc + svec
                done = jnp.where(any_, jnp.int32(1), done)
                return acc, done, bin_, below, binval

            z = jnp.int32(0)
            _, _, bin_, below, binval = lax.fori_loop(
                0, nb // 16, scan_body, (z, z, z, z, z)
            )
            return bin_, below, binval

        def dma(ch, b, sem):
            return pltpu.make_async_copy(
                loss_hbm.at[pl.ds(base + ch * _CHUNK, _CHUNK)],
                data_v.at[b],
                sem,
            )

        def stream_loop(process_batch):
            dma(0, 0, sem0).start()
            dma(1, 1, sem1).start()

            def gbody(g, _):
                for b, sem in ((0, sem0), (1, sem1)):
                    ch = g * 2 + b
                    dma(ch, b, sem).wait()

                    def vbody(j, _):
                        vals = [
                            data_v[b, pl.ds((j * _U + u) * 16, 16)]
                            for u in range(_U)
                        ]
                        process_batch(vals)
                        return 0

                    lax.fori_loop(0, _CHUNK // 16 // _U, vbody, 0)

                    @pl.when(ch + 2 < nch)
                    def _():
                        dma(ch + 2, b, sem).start()

                return 0

            lax.fori_loop(0, nch // 2, gbody, 0)

        r = jnp.int32(num)
        cnt = jnp.int32(N)

        zero_hist(2048, False)

        def p1(vals):
            bins = []
            for v in vals:
                b = plsc.bitcast(v, jnp.int32)
                bins.append(lax.shift_right_logical(b, 20))
            for binv in bins:
                plsc.addupdate_scatter(hist_v, [iota, binv], ones)

        stream_loop(p1)
        merge_tiles(2048, False)
        b1, below, binval = scan(2048, cnt - r + 1)
        r = r - (cnt - below - binval)
        cnt = binval

        zero_hist(1024, False)

        def p2(vals):
            bins = []
            for v in vals:
                b = plsc.bitcast(v, jnp.int32)
                keym = lax.shift_right_logical(b, 20) == b1
                binv = lax.shift_right_logical(b, 10) & 1023
                bins.append((binv, keym))
            for binv, keym in bins:
                plsc.addupdate_scatter(hist_v, [iota, binv], ones, mask=keym)

        stream_loop(p2)
        merge_tiles(1024, False)
        b2, below, binval = scan(1024, cnt - r + 1)
        r = r - (cnt - below - binval)
        cnt = binval

        sel21 = (b1 << 10) | b2
        zero_hist(1040, True)

        def p3(vals):
            bins = []
            for v in vals:
                b = plsc.bitcast(v, jnp.int32)
                top21 = lax.shift_right_logical(b, 10)
                m = top21 >= sel21
                binv = jnp.where(top21 == sel21, b & 1023, jnp.int32(1024))
                bins.append((binv, m, v))
            for binv, m, v in bins:
                plsc.addupdate_scatter(hist_v, [iota, binv], ones, mask=m)
            for binv, m, v in bins:
                plsc.addupdate_scatter(vhist_v, [iota, binv], v, mask=m)

        stream_loop(p3)
        merge_tiles(1040, True)
        b3, _, _ = scan(1024, cnt - r + 1)

        def suf_body(jj, carry):
            tc, pc, ts, ps = carry
            vec = merged_v[pl.ds(jj * 16, 16)]
            vvec = mergedv_v[pl.ds(jj * 16, 16)]
            inpref = (jj * 16 + iota) < b3
            tc = tc + vec
            pc = pc + jnp.where(inpref, vec, 0)
            ts = ts + vvec
            ps = ps + jnp.where(inpref, vvec, jnp.float32(0))
            return tc, pc, ts, ps

        tc, pc, ts, ps = lax.fori_loop(
            0, 1040 // 16, suf_body, (zeros16, zeros16, zf16, zf16)
        )
        c_tot = (jnp.sum(tc) - jnp.sum(pc)).astype(jnp.float32)
        s_tot = jnp.sum(ts) - jnp.sum(ps)

        @pl.when(wid == 0)
        def _():
            num_vec = jnp.full((16,), s_tot, jnp.float32)
            den_vec = jnp.full((16,), c_tot, jnp.float32)
            outbuf_v[...] = num_vec / den_vec
            pltpu.sync_copy(outbuf_v, out_hbm)

    return pl.kernel(
        body,
        out_type=jax.ShapeDtypeStruct((16,), jnp.float32),
        mesh=mesh,
        compiler_params=pltpu.CompilerParams(needs_layout_passes=False),
        scratch_types=[
            pltpu.VMEM((2, _CHUNK), jnp.float32),
            pltpu.VMEM((16, _NBINS), jnp.int32),
            pltpu.VMEM((16, _NBINS), jnp.float32),
            pltpu.VMEM((_NBINS,), jnp.int32),
            pltpu.VMEM((_NBINS,), jnp.float32),
            pltpu.VMEM((16,), jnp.float32),
            pltpu.VMEM_SHARED((_NSUB, _NBINS), jnp.int32),
            pltpu.VMEM_SHARED((_NSUB, _NBINS), jnp.float32),
            pltpu.SemaphoreType.DMA,
            pltpu.SemaphoreType.DMA,
        ],
    )


def kernel(pred, target):
    B, C, H, W = pred.shape
    N = B * H * W
    num = int(_K * B * H * W)
    loss = _nll(pred, target)
    out = _sc_select(N, num)(loss.reshape(N))
    return out[0]


# speedup vs baseline: 14.1874x; 1.1344x over previous
import functools

import jax
import jax.numpy as jnp
from jax import lax
from jax.experimental import pallas as pl
from jax.experimental.pallas import tpu as pltpu
from jax.experimental.pallas import tpu_sc as plsc

_K = 0.15


def _nll_body(pred_ref, tgt_ref, out_ref):
    x = pred_ref[0]
    t = tgt_ref[0]
    m = jnp.max(x, axis=0)
    s = jnp.sum(jnp.exp(x - m[None]), axis=0)
    lse = m + jnp.log(s)
    cls = lax.broadcasted_iota(jnp.int32, x.shape, 0)
    xt = jnp.sum(jnp.where(cls == t[None], x, 0.0), axis=0)
    out_ref[0] = lse - xt


def _nll(pred, target):
    B, C, H, W = pred.shape
    R = 64
    return pl.pallas_call(
        _nll_body,
        grid=(B, H // R),
        in_specs=[
            pl.BlockSpec((1, C, R, W), lambda b, i: (b, 0, i, 0)),
            pl.BlockSpec((1, R, W), lambda b, i: (b, i, 0)),
        ],
        out_specs=pl.BlockSpec((1, R, W), lambda b, i: (b, i, 0)),
        out_shape=jax.ShapeDtypeStruct((B, H, W), jnp.float32),
    )(pred, target)


_NSUB = 16
_CHUNK = 16384
_NBINS = 2048
_U = 8


@functools.lru_cache(maxsize=None)
def _sc_select(N, num):
    per = N // _NSUB
    nch = per // _CHUNK
    assert per % _CHUNK == 0
    mesh = plsc.VectorSubcoreMesh(
        core_axis_name="c", subcore_axis_name="s", num_cores=1
    )

    def body(loss_hbm, out_hbm, data_v, hist_v, vhist_v, merged_v, mergedv_v,
             outbuf_v, sh_hist, sh_vhist, sem0, sem1):
        wid = lax.axis_index("s")
        base = wid * per
        iota = lax.iota(jnp.int32, 16)
        ones = jnp.ones((16,), jnp.int32)
        zeros16 = jnp.zeros((16,), jnp.int32)
        zf16 = jnp.zeros((16,), jnp.float32)

        def zero_hist(width, with_values):
            def zbody(cc, _):
                for l in range(16):
                    hist_v[l, pl.ds(cc * 16, 16)] = zeros16
                    if with_values:
                        vhist_v[l, pl.ds(cc * 16, 16)] = zf16
                return 0

            lax.fori_loop(0, width // 16, zbody, 0)

        def lane_merge(src, dst, width):
            def mbody(cc, _):
                acc = src[0, pl.ds(cc * 16, 16)]
                for l in range(1, 16):
                    acc = acc + src[l, pl.ds(cc * 16, 16)]
                dst[pl.ds(cc * 16, 16)] = acc
                return 0

            lax.fori_loop(0, width // 16, mbody, 0)

        def merge_tiles(width, with_values):
            lane_merge(hist_v, merged_v, width)
            if with_values:
                lane_merge(vhist_v, mergedv_v, width)
            pltpu.sync_copy(merged_v, sh_hist.at[wid])
            if with_values:
                pltpu.sync_copy(mergedv_v, sh_vhist.at[wid])
            plsc.subcore_barrier()
            pltpu.sync_copy(sh_hist, hist_v)
            if with_values:
                pltpu.sync_copy(sh_vhist, vhist_v)
            plsc.subcore_barrier()
            lane_merge(hist_v, merged_v, width)
            if with_values:
                lane_merge(vhist_v, mergedv_v, width)

        def scan(nb, rprime):
            def scan_body(jj, carry):
                acc, done, bin_, below, binval = carry
                vec = merged_v[pl.ds(jj * 16, 16)]
                svec = jnp.sum(vec)
                cum = plsc.cumsum(vec)
                a = acc + cum
                crossm = a >= rprime
                any_ = (acc + svec) >= rprime
                i0 = plsc.all_reduce_ffs(crossm)
                sel = iota == i0
                v_i0 = jnp.sum(jnp.where(sel, vec, 0))
                a_i0 = jnp.sum(jnp.where(sel, a, 0))
                lane = jnp.sum(jnp.where(sel, iota, 0))
                upd = jnp.logical_and(any_, done == 0)
                bin_ = jnp.where(upd, jj * 16 + lane, bin_)
                below = jnp.where(upd, a_i0 - v_i0, below)
                binval = jnp.where(upd, v_i0, binval)
                acc = acc + svec
                done = jnp.where(any_, jnp.int32(1), done)
                return acc, done, bin_, below, binval

            z = jnp.int32(0)
            _, _, bin_, below, binval = lax.fori_loop(
                0, nb // 16, scan_body, (z, z, z, z, z)
            )
            return bin_, below, binval

        def dma(ch, b, sem):
            return pltpu.make_async_copy(
                loss_hbm.at[pl.ds(base + ch * _CHUNK, _CHUNK)],
                data_v.at[b],
                sem,
            )

        def stream_loop(process_batch):
            dma(0, 0, sem0).start()
            dma(1, 1, sem1).start()

            def gbody(g, _):
                for b, sem in ((0, sem0), (1, sem1)):
                    ch = g * 2 + b
                    dma(ch, b, sem).wait()

                    def vbody(j, _):
                        vals = [
                            data_v[b, pl.ds((j * _U + u) * 16, 16)]
                            for u in range(_U)
                        ]
                        process_batch(vals)
                        return 0

                    lax.fori_loop(0, _CHUNK // 16 // _U, vbody, 0)

                    @pl.when(ch + 2 < nch)
                    def _():
                        dma(ch + 2, b, sem).start()

                return 0

            lax.fori_loop(0, nch // 2, gbody, 0)

        r = jnp.int32(num)
        cnt = jnp.int32(N)

        zero_hist(2048, False)

        def p1(vals):
            bins = []
            for v in vals:
                b = plsc.bitcast(v, jnp.int32)
                bins.append(lax.shift_right_logical(b, 20))
            for binv in bins:
                plsc.addupdate_scatter(hist_v, [iota, binv], ones)

        stream_loop(p1)
        merge_tiles(2048, False)
        b1, below, binval = scan(2048, cnt - r + 1)
        r = r - (cnt - below - binval)
        cnt = binval

        zero_hist(1024, False)

        def p2(vals):
            bins = []
            for v in vals:
                b = plsc.bitcast(v, jnp.int32)
                keym = lax.shift_right_logical(b, 20) == b1
                binv = lax.shift_right_logical(b, 10) & 1023
                bins.append((binv, keym))
            for binv, keym in bins:
                plsc.addupdate_scatter(hist_v, [iota, binv], ones, mask=keym)

        stream_loop(p2)
        merge_tiles(1024, False)
        b2, below, binval = scan(1024, cnt - r + 1)
        r = r - (cnt - below - binval)
        cnt = binval

        sel21 = (b1 << 10) | b2
        zero_hist(1040, True)

        def p3(vals):
            bins = []
            for v in vals:
                b = plsc.bitcast(v, jnp.int32)
                top21 = lax.shift_right_logical(b, 10)
                m = top21 >= sel21
                binv = jnp.where(top21 == sel21, b & 1023, jnp.int32(1024))
                bins.append((binv, m, v))
            for binv, m, v in bins:
                plsc.addupdate_scatter(hist_v, [iota, binv], ones, mask=m)
            for binv, m, v in bins:
                plsc.addupdate_scatter(vhist_v, [iota, binv], v, mask=m)

        stream_loop(p3)
        merge_tiles(1040, True)
        b3, _, _ = scan(1024, cnt - r + 1)

        def suf_body(jj, carry):
            tc, pc, ts, ps = carry
            vec = merged_v[pl.ds(jj * 16, 16)]
            vvec = mergedv_v[pl.ds(jj * 16, 16)]
            inpref = (jj * 16 + iota) < b3
            tc = tc + vec
            pc = pc + jnp.where(inpref, vec, 0)
            ts = ts + vvec
            ps = ps + jnp.where(inpref, vvec, jnp.float32(0))
            return tc, pc, ts, ps

        tc, pc, ts, ps = lax.fori_loop(
            0, 1040 // 16, suf_body, (zeros16, zeros16, zf16, zf16)
        )
        c_tot = (jnp.sum(tc) - jnp.sum(pc)).astype(jnp.float32)
        s_tot = jnp.sum(ts) - jnp.sum(ps)

        @pl.when(wid == 0)
        def _():
            num_vec = jnp.full((16,), s_tot, jnp.float32)
            den_vec = jnp.full((16,), c_tot, jnp.float32)
            outbuf_v[...] = num_vec / den_vec
            pltpu.sync_copy(outbuf_v, out_hbm)

    return pl.kernel(
        body,
        out_type=jax.ShapeDtypeStruct((16,), jnp.float32),
        mesh=mesh,
        compiler_params=pltpu.CompilerParams(needs_layout_passes=False),
        scratch_types=[
            pltpu.VMEM((2, _CHUNK), jnp.float32),
            pltpu.VMEM((16, _NBINS), jnp.int32),
            pltpu.VMEM((16, _NBINS), jnp.float32),
            pltpu.VMEM((_NBINS,), jnp.int32),
            pltpu.VMEM((_NBINS,), jnp.float32),
            pltpu.VMEM((16,), jnp.float32),
            pltpu.VMEM_SHARED((_NSUB, _NBINS), jnp.int32),
            pltpu.VMEM_SHARED((_NSUB, _NBINS), jnp.float32),
            pltpu.SemaphoreType.DMA,
            pltpu.SemaphoreType.DMA,
        ],
    )


def kernel(pred, target):
    B, C, H, W = pred.shape
    N = B * H * W
    num = int(_K * B * H * W)
    loss = _nll(pred, target)
    out = _sc_select(N, num)(loss.reshape(N))
    return out[0]


# speedup vs baseline: 15.3000x; 1.0784x over previous
import functools

import jax
import jax.numpy as jnp
from jax import lax
from jax.experimental import pallas as pl
from jax.experimental.pallas import tpu as pltpu
from jax.experimental.pallas import tpu_sc as plsc

_K = 0.15


def _nll_body(pred_ref, tgt_ref, out_ref):
    x = pred_ref[0]
    t = tgt_ref[0]
    m = jnp.max(x, axis=0)
    s = jnp.sum(jnp.exp(x - m[None]), axis=0)
    lse = m + jnp.log(s)
    cls = lax.broadcasted_iota(jnp.int32, x.shape, 0)
    xt = jnp.sum(jnp.where(cls == t[None], x, 0.0), axis=0)
    out_ref[0] = lse - xt


def _nll(pred, target):
    B, C, H, W = pred.shape
    R = 128
    return pl.pallas_call(
        _nll_body,
        grid=(B, H // R),
        in_specs=[
            pl.BlockSpec((1, C, R, W), lambda b, i: (b, 0, i, 0)),
            pl.BlockSpec((1, R, W), lambda b, i: (b, i, 0)),
        ],
        out_specs=pl.BlockSpec((1, R, W), lambda b, i: (b, i, 0)),
        out_shape=jax.ShapeDtypeStruct((B, H, W), jnp.float32),
    )(pred, target)


_NSUB = 16
_CHUNK = 16384
_NBINS = 2048
_U = 8


@functools.lru_cache(maxsize=None)
def _sc_select(N, num):
    per = N // _NSUB
    nch = per // _CHUNK
    assert per % _CHUNK == 0
    mesh = plsc.VectorSubcoreMesh(
        core_axis_name="c", subcore_axis_name="s", num_cores=1
    )

    def body(loss_hbm, out_hbm, data_v, hist_v, vhist_v, merged_v, mergedv_v,
             outbuf_v, sh_hist, sh_vhist, sem0, sem1):
        wid = lax.axis_index("s")
        base = wid * per
        iota = lax.iota(jnp.int32, 16)
        ones = jnp.ones((16,), jnp.int32)
        zeros16 = jnp.zeros((16,), jnp.int32)
        zf16 = jnp.zeros((16,), jnp.float32)

        def zero_hist(width, with_values):
            def zbody(cc, _):
                for l in range(16):
                    hist_v[l, pl.ds(cc * 16, 16)] = zeros16
                    if with_values:
                        vhist_v[l, pl.ds(cc * 16, 16)] = zf16
                return 0

            lax.fori_loop(0, width // 16, zbody, 0)

        def lane_merge(src, dst, width):
            def mbody(cc, _):
                acc = src[0, pl.ds(cc * 16, 16)]
                for l in range(1, 16):
                    acc = acc + src[l, pl.ds(cc * 16, 16)]
                dst[pl.ds(cc * 16, 16)] = acc
                return 0

            lax.fori_loop(0, width // 16, mbody, 0)

        def merge_tiles(width, with_values):
            lane_merge(hist_v, merged_v, width)
            if with_values:
                lane_merge(vhist_v, mergedv_v, width)
            pltpu.sync_copy(merged_v, sh_hist.at[wid])
            if with_values:
                pltpu.sync_copy(mergedv_v, sh_vhist.at[wid])
            plsc.subcore_barrier()
            pltpu.sync_copy(sh_hist, hist_v)
            if with_values:
                pltpu.sync_copy(sh_vhist, vhist_v)
            plsc.subcore_barrier()
            lane_merge(hist_v, merged_v, width)
            if with_values:
                lane_merge(vhist_v, mergedv_v, width)

        def scan(nb, rprime):
            def scan_body(jj, carry):
                acc, done, bin_, below, binval = carry
                vec = merged_v[pl.ds(jj * 16, 16)]
                svec = jnp.sum(vec)
                cum = plsc.cumsum(vec)
                a = acc + cum
                crossm = a >= rprime
                any_ = (acc + svec) >= rprime
                i0 = plsc.all_reduce_ffs(crossm)
                sel = iota == i0
                v_i0 = jnp.sum(jnp.where(sel, vec, 0))
                a_i0 = jnp.sum(jnp.where(sel, a, 0))
                lane = jnp.sum(jnp.where(sel, iota, 0))
                upd = jnp.logical_and(any_, done == 0)
                bin_ = jnp.where(upd, jj * 16 + lane, bin_)
                below = jnp.where(upd, a_i0 - v_i0, below)
                binval = jnp.where(upd, v_i0, binval)
                acc = acc + svec
                done = jnp.where(any_, jnp.int32(1), done)
                return acc, done, bin_, below, binval

            z = jnp.int32(0)
            _, _, bin_, below, binval = lax.fori_loop(
                0, nb // 16, scan_body, (z, z, z, z, z)
            )
            return bin_, below, binval

        def dma(ch, b, sem):
            return pltpu.make_async_copy(
                loss_hbm.at[pl.ds(base + ch * _CHUNK, _CHUNK)],
                data_v.at[b],
                sem,
            )

        def stream_loop(process_batch):
            dma(0, 0, sem0).start()
            dma(1, 1, sem1).start()

            def gbody(g, _):
                for b, sem in ((0, sem0), (1, sem1)):
                    ch = g * 2 + b
                    dma(ch, b, sem).wait()

                    def vbody(j, _):
                        vals = [
                            data_v[b, pl.ds((j * _U + u) * 16, 16)]
                            for u in range(_U)
                        ]
                        process_batch(vals)
                        return 0

                    lax.fori_loop(0, _CHUNK // 16 // _U, vbody, 0)

                    @pl.when(ch + 2 < nch)
                    def _():
                        dma(ch + 2, b, sem).start()

                return 0

            lax.fori_loop(0, nch // 2, gbody, 0)

        r = jnp.int32(num)
        cnt = jnp.int32(N)

        zero_hist(2048, False)

        def p1(vals):
            bins = []
            for v in vals:
                b = plsc.bitcast(v, jnp.int32)
                bins.append(lax.shift_right_logical(b, 20))
            for binv in bins:
                plsc.addupdate_scatter(hist_v, [iota, binv], ones)

        stream_loop(p1)
        merge_tiles(2048, False)
        b1, below, binval = scan(2048, cnt - r + 1)
        r = r - (cnt - below - binval)
        cnt = binval

        zero_hist(1024, False)

        def p2(vals):
            bins = []
            for v in vals:
                b = plsc.bitcast(v, jnp.int32)
                keym = lax.shift_right_logical(b, 20) == b1
                binv = lax.shift_right_logical(b, 10) & 1023
                bins.append((binv, keym))
            for binv, keym in bins:
                plsc.addupdate_scatter(hist_v, [iota, binv], ones, mask=keym)

        stream_loop(p2)
        merge_tiles(1024, False)
        b2, below, binval = scan(1024, cnt - r + 1)
        r = r - (cnt - below - binval)
        cnt = binval

        sel21 = (b1 << 10) | b2
        zero_hist(1040, True)

        def p3(vals):
            bins = []
            for v in vals:
                b = plsc.bitcast(v, jnp.int32)
                top21 = lax.shift_right_logical(b, 10)
                m = top21 >= sel21
                binv = jnp.where(top21 == sel21, b & 1023, jnp.int32(1024))
                bins.append((binv, m, v))
            for binv, m, v in bins:
                plsc.addupdate_scatter(hist_v, [iota, binv], ones, mask=m)
            for binv, m, v in bins:
                plsc.addupdate_scatter(vhist_v, [iota, binv], v, mask=m)

        stream_loop(p3)
        merge_tiles(1040, True)
        b3, _, _ = scan(1024, cnt - r + 1)

        def suf_body(jj, carry):
            tc, pc, ts, ps = carry
            vec = merged_v[pl.ds(jj * 16, 16)]
            vvec = mergedv_v[pl.ds(jj * 16, 16)]
            inpref = (jj * 16 + iota) < b3
            tc = tc + vec
            pc = pc + jnp.where(inpref, vec, 0)
            ts = ts + vvec
            ps = ps + jnp.where(inpref, vvec, jnp.float32(0))
            return tc, pc, ts, ps

        tc, pc, ts, ps = lax.fori_loop(
            0, 1040 // 16, suf_body, (zeros16, zeros16, zf16, zf16)
        )
        c_tot = (jnp.sum(tc) - jnp.sum(pc)).astype(jnp.float32)
        s_tot = jnp.sum(ts) - jnp.sum(ps)

        @pl.when(wid == 0)
        def _():
            num_vec = jnp.full((16,), s_tot, jnp.float32)
            den_vec = jnp.full((16,), c_tot, jnp.float32)
            outbuf_v[...] = num_vec / den_vec
            pltpu.sync_copy(outbuf_v, out_hbm)

    return pl.kernel(
        body,
        out_type=jax.ShapeDtypeStruct((16,), jnp.float32),
        mesh=mesh,
        compiler_params=pltpu.CompilerParams(needs_layout_passes=False),
        scratch_types=[
            pltpu.VMEM((2, _CHUNK), jnp.float32),
            pltpu.VMEM((16, _NBINS), jnp.int32),
            pltpu.VMEM((16, _NBINS), jnp.float32),
            pltpu.VMEM((_NBINS,), jnp.int32),
            pltpu.VMEM((_NBINS,), jnp.float32),
            pltpu.VMEM((16,), jnp.float32),
            pltpu.VMEM_SHARED((_NSUB, _NBINS), jnp.int32),
            pltpu.VMEM_SHARED((_NSUB, _NBINS), jnp.float32),
            pltpu.SemaphoreType.DMA,
            pltpu.SemaphoreType.DMA,
        ],
    )


def kernel(pred, target):
    B, C, H, W = pred.shape
    N = B * H * W
    num = int(_K * B * H * W)
    loss = _nll(pred, target)
    out = _sc_select(N, num)(loss.reshape(N))
    return out[0]


# speedup vs baseline: 15.8646x; 1.0369x over previous
import functools

import jax
import jax.numpy as jnp
from jax import lax
from jax.experimental import pallas as pl
from jax.experimental.pallas import tpu as pltpu
from jax.experimental.pallas import tpu_sc as plsc

_K = 0.15


def _nll_body(pred_ref, tgt_ref, out_ref):
    x = pred_ref[0]
    t = tgt_ref[0]
    m = jnp.max(x, axis=0)
    s = jnp.sum(jnp.exp(x - m[None]), axis=0)
    lse = m + jnp.log(s)
    cls = lax.broadcasted_iota(jnp.int32, x.shape, 0)
    xt = jnp.sum(jnp.where(cls == t[None], x, 0.0), axis=0)
    out_ref[0] = lse - xt


def _nll(pred, target):
    B, C, H, W = pred.shape
    R = 256
    return pl.pallas_call(
        _nll_body,
        grid=(B, H // R),
        in_specs=[
            pl.BlockSpec((1, C, R, W), lambda b, i: (b, 0, i, 0)),
            pl.BlockSpec((1, R, W), lambda b, i: (b, i, 0)),
        ],
        out_specs=pl.BlockSpec((1, R, W), lambda b, i: (b, i, 0)),
        out_shape=jax.ShapeDtypeStruct((B, H, W), jnp.float32),
    )(pred, target)


_NSUB = 16
_CHUNK = 16384
_NBINS = 2048
_U = 8


@functools.lru_cache(maxsize=None)
def _sc_select(N, num):
    per = N // _NSUB
    nch = per // _CHUNK
    assert per % _CHUNK == 0
    mesh = plsc.VectorSubcoreMesh(
        core_axis_name="c", subcore_axis_name="s", num_cores=1
    )

    def body(loss_hbm, out_hbm, data_v, hist_v, vhist_v, merged_v, mergedv_v,
             outbuf_v, sh_hist, sh_vhist, sem0, sem1):
        wid = lax.axis_index("s")
        base = wid * per
        iota = lax.iota(jnp.int32, 16)
        ones = jnp.ones((16,), jnp.int32)
        zeros16 = jnp.zeros((16,), jnp.int32)
        zf16 = jnp.zeros((16,), jnp.float32)

        def zero_hist(width, with_values):
            def zbody(cc, _):
                for l in range(16):
                    hist_v[l, pl.ds(cc * 16, 16)] = zeros16
                    if with_values:
                        vhist_v[l, pl.ds(cc * 16, 16)] = zf16
                return 0

            lax.fori_loop(0, width // 16, zbody, 0)

        def lane_merge(src, dst, width):
            def mbody(cc, _):
                acc = src[0, pl.ds(cc * 16, 16)]
                for l in range(1, 16):
                    acc = acc + src[l, pl.ds(cc * 16, 16)]
                dst[pl.ds(cc * 16, 16)] = acc
                return 0

            lax.fori_loop(0, width // 16, mbody, 0)

        def merge_tiles(width, with_values):
            lane_merge(hist_v, merged_v, width)
            if with_values:
                lane_merge(vhist_v, mergedv_v, width)
            pltpu.sync_copy(merged_v, sh_hist.at[wid])
            if with_values:
                pltpu.sync_copy(mergedv_v, sh_vhist.at[wid])
            plsc.subcore_barrier()
            pltpu.sync_copy(sh_hist, hist_v)
            if with_values:
                pltpu.sync_copy(sh_vhist, vhist_v)
            plsc.subcore_barrier()
            lane_merge(hist_v, merged_v, width)
            if with_values:
                lane_merge(vhist_v, mergedv_v, width)

        def scan(nb, rprime):
            def scan_body(jj, carry):
                acc, done, bin_, below, binval = carry
                vec = merged_v[pl.ds(jj * 16, 16)]
                svec = jnp.sum(vec)
                cum = plsc.cumsum(vec)
                a = acc + cum
                crossm = a >= rprime
                any_ = (acc + svec) >= rprime
                i0 = plsc.all_reduce_ffs(crossm)
                sel = iota == i0
                v_i0 = jnp.sum(jnp.where(sel, vec, 0))
                a_i0 = jnp.sum(jnp.where(sel, a, 0))
                lane = jnp.sum(jnp.where(sel, iota, 0))
                upd = jnp.logical_and(any_, done == 0)
                bin_ = jnp.where(upd, jj * 16 + lane, bin_)
                below = jnp.where(upd, a_i0 - v_i0, below)
                binval = jnp.where(upd, v_i0, binval)
                acc = acc + svec
                done = jnp.where(any_, jnp.int32(1), done)
                return acc, done, bin_, below, binval

            z = jnp.int32(0)
            _, _, bin_, below, binval = lax.fori_loop(
                0, nb // 16, scan_body, (z, z, z, z, z)
            )
            return bin_, below, binval

        def dma(ch, b, sem):
            return pltpu.make_async_copy(
                loss_hbm.at[pl.ds(base + ch * _CHUNK, _CHUNK)],
                data_v.at[b],
                sem,
            )

        def stream_loop(process_batch):
            dma(0, 0, sem0).start()
            dma(1, 1, sem1).start()

            def gbody(g, _):
                for b, sem in ((0, sem0), (1, sem1)):
                    ch = g * 2 + b
                    dma(ch, b, sem).wait()

                    def vbody(j, _):
                        vals = [
                            data_v[b, pl.ds((j * _U + u) * 16, 16)]
                            for u in range(_U)
                        ]
                        process_batch(vals)
                        return 0

                    lax.fori_loop(0, _CHUNK // 16 // _U, vbody, 0)

                    @pl.when(ch + 2 < nch)
                    def _():
                        dma(ch + 2, b, sem).start()

                return 0

            lax.fori_loop(0, nch // 2, gbody, 0)

        r = jnp.int32(num)
        cnt = jnp.int32(N)

        zero_hist(2048, False)

        def p1(vals):
            bins = []
            for v in vals:
                b = plsc.bitcast(v, jnp.int32)
                bins.append(lax.shift_right_logical(b, 20))
            for binv in bins:
                plsc.addupdate_scatter(hist_v, [iota, binv], ones)

        stream_loop(p1)
        merge_tiles(2048, False)
        b1, below, binval = scan(2048, cnt - r + 1)
        r = r - (cnt - below - binval)
        cnt = binval

        zero_hist(1024, False)

        def p2(vals):
            bins = []
            for v in vals:
                b = plsc.bitcast(v, jnp.int32)
                keym = lax.shift_right_logical(b, 20) == b1
                binv = lax.shift_right_logical(b, 10) & 1023
                bins.append((binv, keym))
            for binv, keym in bins:
                plsc.addupdate_scatter(hist_v, [iota, binv], ones, mask=keym)

        stream_loop(p2)
        merge_tiles(1024, False)
        b2, below, binval = scan(1024, cnt - r + 1)
        r = r - (cnt - below - binval)
        cnt = binval

        sel21 = (b1 << 10) | b2
        zero_hist(1040, True)

        def p3(vals):
            bins = []
            for v in vals:
                b = plsc.bitcast(v, jnp.int32)
                top21 = lax.shift_right_logical(b, 10)
                m = top21 >= sel21
                binv = jnp.where(top21 == sel21, b & 1023, jnp.int32(1024))
                bins.append((binv, m, v))
            for binv, m, v in bins:
                plsc.addupdate_scatter(hist_v, [iota, binv], ones, mask=m)
            for binv, m, v in bins:
                plsc.addupdate_scatter(vhist_v, [iota, binv], v, mask=m)

        stream_loop(p3)
        merge_tiles(1040, True)
        b3, _, _ = scan(1024, cnt - r + 1)

        def suf_body(jj, carry):
            tc, pc, ts, ps = carry
            vec = merged_v[pl.ds(jj * 16, 16)]
            vvec = mergedv_v[pl.ds(jj * 16, 16)]
            inpref = (jj * 16 + iota) < b3
            tc = tc + vec
            pc = pc + jnp.where(inpref, vec, 0)
            ts = ts + vvec
            ps = ps + jnp.where(inpref, vvec, jnp.float32(0))
            return tc, pc, ts, ps

        tc, pc, ts, ps = lax.fori_loop(
            0, 1040 // 16, suf_body, (zeros16, zeros16, zf16, zf16)
        )
        c_tot = (jnp.sum(tc) - jnp.sum(pc)).astype(jnp.float32)
        s_tot = jnp.sum(ts) - jnp.sum(ps)

        @pl.when(wid == 0)
        def _():
            num_vec = jnp.full((16,), s_tot, jnp.float32)
            den_vec = jnp.full((16,), c_tot, jnp.float32)
            outbuf_v[...] = num_vec / den_vec
            pltpu.sync_copy(outbuf_v, out_hbm)

    return pl.kernel(
        body,
        out_type=jax.ShapeDtypeStruct((16,), jnp.float32),
        mesh=mesh,
        compiler_params=pltpu.CompilerParams(needs_layout_passes=False),
        scratch_types=[
            pltpu.VMEM((2, _CHUNK), jnp.float32),
            pltpu.VMEM((16, _NBINS), jnp.int32),
            pltpu.VMEM((16, _NBINS), jnp.float32),
            pltpu.VMEM((_NBINS,), jnp.int32),
            pltpu.VMEM((_NBINS,), jnp.float32),
            pltpu.VMEM((16,), jnp.float32),
            pltpu.VMEM_SHARED((_NSUB, _NBINS), jnp.int32),
            pltpu.VMEM_SHARED((_NSUB, _NBINS), jnp.float32),
            pltpu.SemaphoreType.DMA,
            pltpu.SemaphoreType.DMA,
        ],
    )


def kernel(pred, target):
    B, C, H, W = pred.shape
    N = B * H * W
    num = int(_K * B * H * W)
    loss = _nll(pred, target)
    out = _sc_select(N, num)(loss.reshape(N))
    return out[0]


# speedup vs baseline: 15.9605x; 1.0060x over previous
import functools

import jax
import jax.numpy as jnp
from jax import lax
from jax.experimental import pallas as pl
from jax.experimental.pallas import tpu as pltpu
from jax.experimental.pallas import tpu_sc as plsc

_K = 0.15


def _nll_body(pred_ref, tgt_ref, out_ref):
    x = pred_ref[0]
    t = tgt_ref[0]
    m = jnp.max(x, axis=0)
    s = jnp.sum(jnp.exp(x - m[None]), axis=0)
    lse = m + jnp.log(s)
    cls = lax.broadcasted_iota(jnp.int32, x.shape, 0)
    xt = jnp.sum(jnp.where(cls == t[None], x, 0.0), axis=0)
    out_ref[0] = lse - xt


def _nll(pred, target):
    B, C, H, W = pred.shape
    R = 512
    return pl.pallas_call(
        _nll_body,
        grid=(B, H // R),
        in_specs=[
            pl.BlockSpec((1, C, R, W), lambda b, i: (b, 0, i, 0)),
            pl.BlockSpec((1, R, W), lambda b, i: (b, i, 0)),
        ],
        out_specs=pl.BlockSpec((1, R, W), lambda b, i: (b, i, 0)),
        out_shape=jax.ShapeDtypeStruct((B, H, W), jnp.float32),
    )(pred, target)


_NSUB = 16
_CHUNK = 16384
_NBINS = 2048
_U = 8


@functools.lru_cache(maxsize=None)
def _sc_select(N, num):
    per = N // _NSUB
    nch = per // _CHUNK
    assert per % _CHUNK == 0
    mesh = plsc.VectorSubcoreMesh(
        core_axis_name="c", subcore_axis_name="s", num_cores=1
    )

    def body(loss_hbm, out_hbm, data_v, hist_v, vhist_v, merged_v, mergedv_v,
             outbuf_v, sh_hist, sh_vhist, sem0, sem1):
        wid = lax.axis_index("s")
        base = wid * per
        iota = lax.iota(jnp.int32, 16)
        ones = jnp.ones((16,), jnp.int32)
        zeros16 = jnp.zeros((16,), jnp.int32)
        zf16 = jnp.zeros((16,), jnp.float32)

        def zero_hist(width, with_values):
            def zbody(cc, _):
                for l in range(16):
                    hist_v[l, pl.ds(cc * 16, 16)] = zeros16
                    if with_values:
                        vhist_v[l, pl.ds(cc * 16, 16)] = zf16
                return 0

            lax.fori_loop(0, width // 16, zbody, 0)

        def lane_merge(src, dst, width):
            def mbody(cc, _):
                acc = src[0, pl.ds(cc * 16, 16)]
                for l in range(1, 16):
                    acc = acc + src[l, pl.ds(cc * 16, 16)]
                dst[pl.ds(cc * 16, 16)] = acc
                return 0

            lax.fori_loop(0, width // 16, mbody, 0)

        def merge_tiles(width, with_values):
            lane_merge(hist_v, merged_v, width)
            if with_values:
                lane_merge(vhist_v, mergedv_v, width)
            pltpu.sync_copy(merged_v, sh_hist.at[wid])
            if with_values:
                pltpu.sync_copy(mergedv_v, sh_vhist.at[wid])
            plsc.subcore_barrier()
            pltpu.sync_copy(sh_hist, hist_v)
            if with_values:
                pltpu.sync_copy(sh_vhist, vhist_v)
            plsc.subcore_barrier()
            lane_merge(hist_v, merged_v, width)
            if with_values:
                lane_merge(vhist_v, mergedv_v, width)

        def scan(nb, rprime):
            def scan_body(jj, carry):
                acc, done, bin_, below, binval = carry
                vec = merged_v[pl.ds(jj * 16, 16)]
                svec = jnp.sum(vec)
                cum = plsc.cumsum(vec)
                a = acc + cum
                crossm = a >= rprime
                any_ = (acc + svec) >= rprime
                i0 = plsc.all_reduce_ffs(crossm)
                sel = iota == i0
                v_i0 = jnp.sum(jnp.where(sel, vec, 0))
                a_i0 = jnp.sum(jnp.where(sel, a, 0))
                lane = jnp.sum(jnp.where(sel, iota, 0))
                upd = jnp.logical_and(any_, done == 0)
                bin_ = jnp.where(upd, jj * 16 + lane, bin_)
                below = jnp.where(upd, a_i0 - v_i0, below)
                binval = jnp.where(upd, v_i0, binval)
                acc = acc + svec
                done = jnp.where(any_, jnp.int32(1), done)
                return acc, done, bin_, below, binval

            z = jnp.int32(0)
            _, _, bin_, below, binval = lax.fori_loop(
                0, nb // 16, scan_body, (z, z, z, z, z)
            )
            return bin_, below, binval

        def dma(ch, b, sem):
            return pltpu.make_async_copy(
                loss_hbm.at[pl.ds(base + ch * _CHUNK, _CHUNK)],
                data_v.at[b],
                sem,
            )

        def stream_loop(process_batch):
            dma(0, 0, sem0).start()
            dma(1, 1, sem1).start()

            def gbody(g, _):
                for b, sem in ((0, sem0), (1, sem1)):
                    ch = g * 2 + b
                    dma(ch, b, sem).wait()

                    def vbody(j, _):
                        vals = [
                            data_v[b, pl.ds((j * _U + u) * 16, 16)]
                            for u in range(_U)
                        ]
                        process_batch(vals)
                        return 0

                    lax.fori_loop(0, _CHUNK // 16 // _U, vbody, 0)

                    @pl.when(ch + 2 < nch)
                    def _():
                        dma(ch + 2, b, sem).start()

                return 0

            lax.fori_loop(0, nch // 2, gbody, 0)

        r = jnp.int32(num)
        cnt = jnp.int32(N)

        zero_hist(2048, False)

        def p1(vals):
            bins = []
            for v in vals:
                b = plsc.bitcast(v, jnp.int32)
                bins.append(lax.shift_right_logical(b, 20))
            for binv in bins:
                plsc.addupdate_scatter(hist_v, [iota, binv], ones)

        stream_loop(p1)
        merge_tiles(2048, False)
        b1, below, binval = scan(2048, cnt - r + 1)
        r = r - (cnt - below - binval)
        cnt = binval

        zero_hist(1024, False)

        def p2(vals):
            bins = []
            for v in vals:
                b = plsc.bitcast(v, jnp.int32)
                keym = lax.shift_right_logical(b, 20) == b1
                binv = lax.shift_right_logical(b, 10) & 1023
                bins.append((binv, keym))
            for binv, keym in bins:
                plsc.addupdate_scatter(hist_v, [iota, binv], ones, mask=keym)

        stream_loop(p2)
        merge_tiles(1024, False)
        b2, below, binval = scan(1024, cnt - r + 1)
        r = r - (cnt - below - binval)
        cnt = binval

        sel21 = (b1 << 10) | b2
        zero_hist(1040, True)

        def p3(vals):
            bins = []
            for v in vals:
                b = plsc.bitcast(v, jnp.int32)
                top21 = lax.shift_right_logical(b, 10)
                m = top21 >= sel21
                binv = jnp.where(top21 == sel21, b & 1023, jnp.int32(1024))
                bins.append((binv, m, v))
            for binv, m, v in bins:
                plsc.addupdate_scatter(hist_v, [iota, binv], ones, mask=m)
            for binv, m, v in bins:
                plsc.addupdate_scatter(vhist_v, [iota, binv], v, mask=m)

        stream_loop(p3)
        merge_tiles(1040, True)
        b3, _, _ = scan(1024, cnt - r + 1)

        def suf_body(jj, carry):
            tc, pc, ts, ps = carry
            vec = merged_v[pl.ds(jj * 16, 16)]
            vvec = mergedv_v[pl.ds(jj * 16, 16)]
            inpref = (jj * 16 + iota) < b3
            tc = tc + vec
            pc = pc + jnp.where(inpref, vec, 0)
            ts = ts + vvec
            ps = ps + jnp.where(inpref, vvec, jnp.float32(0))
            return tc, pc, ts, ps

        tc, pc, ts, ps = lax.fori_loop(
            0, 1040 // 16, suf_body, (zeros16, zeros16, zf16, zf16)
        )
        c_tot = (jnp.sum(tc) - jnp.sum(pc)).astype(jnp.float32)
        s_tot = jnp.sum(ts) - jnp.sum(ps)

        @pl.when(wid == 0)
        def _():
            num_vec = jnp.full((16,), s_tot, jnp.float32)
            den_vec = jnp.full((16,), c_tot, jnp.float32)
            outbuf_v[...] = num_vec / den_vec
            pltpu.sync_copy(outbuf_v, out_hbm)

    return pl.kernel(
        body,
        out_type=jax.ShapeDtypeStruct((16,), jnp.float32),
        mesh=mesh,
        compiler_params=pltpu.CompilerParams(needs_layout_passes=False),
        scratch_types=[
            pltpu.VMEM((2, _CHUNK), jnp.float32),
            pltpu.VMEM((16, _NBINS), jnp.int32),
            pltpu.VMEM((16, _NBINS), jnp.float32),
            pltpu.VMEM((_NBINS,), jnp.int32),
            pltpu.VMEM((_NBINS,), jnp.float32),
            pltpu.VMEM((16,), jnp.float32),
            pltpu.VMEM_SHARED((_NSUB, _NBINS), jnp.int32),
            pltpu.VMEM_SHARED((_NSUB, _NBINS), jnp.float32),
            pltpu.SemaphoreType.DMA,
            pltpu.SemaphoreType.DMA,
        ],
    )


def kernel(pred, target):
    B, C, H, W = pred.shape
    N = B * H * W
    num = int(_K * B * H * W)
    loss = _nll(pred, target)
    out = _sc_select(N, num)(loss.reshape(N))
    return out[0]


# speedup vs baseline: 16.2347x; 1.0172x over previous
import functools

import jax
import jax.numpy as jnp
from jax import lax
from jax.experimental import pallas as pl
from jax.experimental.pallas import tpu as pltpu
from jax.experimental.pallas import tpu_sc as plsc

_K = 0.15


def _nll_body(pred_ref, tgt_ref, out_ref):
    x = pred_ref[0]
    t = tgt_ref[0]
    m = jnp.max(x, axis=0)
    s = jnp.sum(jnp.exp(x - m[None]), axis=0)
    lse = m + jnp.log(s)
    cls = lax.broadcasted_iota(jnp.int32, x.shape, 0)
    xt = jnp.sum(jnp.where(cls == t[None], x, 0.0), axis=0)
    out_ref[0] = lse - xt


def _nll(pred, target):
    B, C, H, W = pred.shape
    R = 512
    return pl.pallas_call(
        _nll_body,
        grid=(B, H // R),
        in_specs=[
            pl.BlockSpec((1, C, R, W), lambda b, i: (b, 0, i, 0)),
            pl.BlockSpec((1, R, W), lambda b, i: (b, i, 0)),
        ],
        out_specs=pl.BlockSpec((1, R, W), lambda b, i: (b, i, 0)),
        out_shape=jax.ShapeDtypeStruct((B, H, W), jnp.float32),
    )(pred, target)


_NSUB = 16
_CHUNK = 16384
_NBINS = 2048
_U = 8


@functools.lru_cache(maxsize=None)
def _sc_select(N, num):
    per = N // _NSUB
    nch = per // _CHUNK
    assert per % _CHUNK == 0
    mesh = plsc.VectorSubcoreMesh(
        core_axis_name="c", subcore_axis_name="s", num_cores=1
    )

    def body(loss_hbm, out_hbm, data_v, hist_v, vhist_v, merged_v, mergedv_v,
             outbuf_v, sh_hist, sh_vhist, sem0, sem1):
        wid = lax.axis_index("s")
        base = wid * per
        iota = lax.iota(jnp.int32, 16)
        ones = jnp.ones((16,), jnp.int32)
        zeros16 = jnp.zeros((16,), jnp.int32)
        zf16 = jnp.zeros((16,), jnp.float32)

        def zero_hist(width, with_values):
            def zbody(cc, _):
                for l in range(16):
                    hist_v[l, pl.ds(cc * 16, 16)] = zeros16
                    if with_values:
                        vhist_v[l, pl.ds(cc * 16, 16)] = zf16
                return 0

            lax.fori_loop(0, width // 16, zbody, 0)

        def lane_merge(src, dst, width):
            def mbody(cc, _):
                acc = src[0, pl.ds(cc * 16, 16)]
                for l in range(1, 16):
                    acc = acc + src[l, pl.ds(cc * 16, 16)]
                dst[pl.ds(cc * 16, 16)] = acc
                return 0

            lax.fori_loop(0, width // 16, mbody, 0)

        def merge_tiles(width, with_values):
            lane_merge(hist_v, merged_v, width)
            if with_values:
                lane_merge(vhist_v, mergedv_v, width)
            pltpu.sync_copy(merged_v, sh_hist.at[wid])
            if with_values:
                pltpu.sync_copy(mergedv_v, sh_vhist.at[wid])
            plsc.subcore_barrier()
            pltpu.sync_copy(sh_hist, hist_v)
            if with_values:
                pltpu.sync_copy(sh_vhist, vhist_v)
            plsc.subcore_barrier()
            lane_merge(hist_v, merged_v, width)
            if with_values:
                lane_merge(vhist_v, mergedv_v, width)

        def scan(nb, rprime):
            def scan_body(jj, carry):
                acc, done, bin_, below, binval = carry
                vec = merged_v[pl.ds(jj * 16, 16)]
                svec = jnp.sum(vec)
                cum = plsc.cumsum(vec)
                a = acc + cum
                crossm = a >= rprime
                any_ = (acc + svec) >= rprime
                i0 = plsc.all_reduce_ffs(crossm)
                sel = iota == i0
                v_i0 = jnp.sum(jnp.where(sel, vec, 0))
                a_i0 = jnp.sum(jnp.where(sel, a, 0))
                lane = jnp.sum(jnp.where(sel, iota, 0))
                upd = jnp.logical_and(any_, done == 0)
                bin_ = jnp.where(upd, jj * 16 + lane, bin_)
                below = jnp.where(upd, a_i0 - v_i0, below)
                binval = jnp.where(upd, v_i0, binval)
                acc = acc + svec
                done = jnp.where(any_, jnp.int32(1), done)
                return acc, done, bin_, below, binval

            z = jnp.int32(0)
            _, _, bin_, below, binval = lax.fori_loop(
                0, nb // 16, scan_body, (z, z, z, z, z)
            )
            return bin_, below, binval

        def dma(ch, b, sem):
            return pltpu.make_async_copy(
                loss_hbm.at[pl.ds(base + ch * _CHUNK, _CHUNK)],
                data_v.at[b],
                sem,
            )

        def stream_loop(process_batch, unroll, first=False, last=False):
            if first:
                dma(0, 0, sem0).start()
                dma(1, 1, sem1).start()

            def gbody(g, _):
                for b, sem in ((0, sem0), (1, sem1)):
                    ch = g * 2 + b
                    dma(ch, b, sem).wait()

                    def vbody(j, _):
                        vals = [
                            data_v[b, pl.ds((j * unroll + u) * 16, 16)]
                            for u in range(unroll)
                        ]
                        process_batch(vals)
                        return 0

                    lax.fori_loop(0, _CHUNK // 16 // unroll, vbody, 0)

                    if last:

                        @pl.when(ch + 2 < nch)
                        def _():
                            dma(ch + 2, b, sem).start()

                    else:

                        @pl.when(ch + 2 < nch)
                        def _():
                            dma(ch + 2, b, sem).start()

                        @pl.when(ch + 2 >= nch)
                        def _():
                            dma(ch + 2 - nch, b, sem).start()

                return 0

            lax.fori_loop(0, nch // 2, gbody, 0)

        r = jnp.int32(num)
        cnt = jnp.int32(N)

        zero_hist(2048, False)

        def p1(vals):
            bins = []
            for v in vals:
                b = plsc.bitcast(v, jnp.int32)
                bins.append(lax.shift_right_logical(b, 20))
            for binv in bins:
                plsc.addupdate_scatter(hist_v, [iota, binv], ones)

        stream_loop(p1, 16, first=True)
        merge_tiles(2048, False)
        b1, below, binval = scan(2048, cnt - r + 1)
        r = r - (cnt - below - binval)
        cnt = binval

        zero_hist(1024, False)

        def p2(vals):
            bins = []
            for v in vals:
                b = plsc.bitcast(v, jnp.int32)
                keym = lax.shift_right_logical(b, 20) == b1
                binv = lax.shift_right_logical(b, 10) & 1023
                bins.append((binv, keym))
            for binv, keym in bins:
                plsc.addupdate_scatter(hist_v, [iota, binv], ones, mask=keym)

        stream_loop(p2, 16)
        merge_tiles(1024, False)
        b2, below, binval = scan(1024, cnt - r + 1)
        r = r - (cnt - below - binval)
        cnt = binval

        sel21 = (b1 << 10) | b2
        zero_hist(1040, True)

        def p3(vals):
            bins = []
            for v in vals:
                b = plsc.bitcast(v, jnp.int32)
                top21 = lax.shift_right_logical(b, 10)
                m = top21 >= sel21
                binv = jnp.where(top21 == sel21, b & 1023, jnp.int32(1024))
                bins.append((binv, m, v))
            for binv, m, v in bins:
                plsc.addupdate_scatter(hist_v, [iota, binv], ones, mask=m)
            for binv, m, v in bins:
                plsc.addupdate_scatter(vhist_v, [iota, binv], v, mask=m)

        stream_loop(p3, 8, last=True)
        merge_tiles(1040, True)
        b3, _, _ = scan(1024, cnt - r + 1)

        def suf_body(jj, carry):
            tc, pc, ts, ps = carry
            vec = merged_v[pl.ds(jj * 16, 16)]
            vvec = mergedv_v[pl.ds(jj * 16, 16)]
            inpref = (jj * 16 + iota) < b3
            tc = tc + vec
            pc = pc + jnp.where(inpref, vec, 0)
            ts = ts + vvec
            ps = ps + jnp.where(inpref, vvec, jnp.float32(0))
            return tc, pc, ts, ps

        tc, pc, ts, ps = lax.fori_loop(
            0, 1040 // 16, suf_body, (zeros16, zeros16, zf16, zf16)
        )
        c_tot = (jnp.sum(tc) - jnp.sum(pc)).astype(jnp.float32)
        s_tot = jnp.sum(ts) - jnp.sum(ps)

        @pl.when(wid == 0)
        def _():
            num_vec = jnp.full((16,), s_tot, jnp.float32)
            den_vec = jnp.full((16,), c_tot, jnp.float32)
            outbuf_v[...] = num_vec / den_vec
            pltpu.sync_copy(outbuf_v, out_hbm)

    return pl.kernel(
        body,
        out_type=jax.ShapeDtypeStruct((16,), jnp.float32),
        mesh=mesh,
        compiler_params=pltpu.CompilerParams(needs_layout_passes=False),
        scratch_types=[
            pltpu.VMEM((2, _CHUNK), jnp.float32),
            pltpu.VMEM((16, _NBINS), jnp.int32),
            pltpu.VMEM((16, _NBINS), jnp.float32),
            pltpu.VMEM((_NBINS,), jnp.int32),
            pltpu.VMEM((_NBINS,), jnp.float32),
            pltpu.VMEM((16,), jnp.float32),
            pltpu.VMEM_SHARED((_NSUB, _NBINS), jnp.int32),
            pltpu.VMEM_SHARED((_NSUB, _NBINS), jnp.float32),
            pltpu.SemaphoreType.DMA,
            pltpu.SemaphoreType.DMA,
        ],
    )


def kernel(pred, target):
    B, C, H, W = pred.shape
    N = B * H * W
    num = int(_K * B * H * W)
    loss = _nll(pred, target)
    out = _sc_select(N, num)(loss.reshape(N))
    return out[0]
